# baseline (speedup 1.0000x reference)
import functools

import jax
import jax.numpy as jnp
from jax.experimental import pallas as pl
from jax.experimental.pallas import tpu as pltpu

N = 10000
IN_CH = 128
D = 32
SPACE = 3
K = 40
NB = 4


def _elu(v):
    return jnp.where(v > 0, v, jnp.exp(jnp.minimum(v, 0.0)) - 1.0)


def _bn_in(h, g, b):
    m = jnp.mean(h, axis=0, keepdims=True)
    v = jnp.mean((h - m) ** 2, axis=0, keepdims=True)
    return (h - m) * jax.lax.rsqrt(v + 1e-5) * g + b


def _pre_body(x_ref, batchf_ref, w1_ref, b1_ref, w2_ref, b2_ref, g1_ref,
              be1_ref, ws_ref, bs_ref, wh_ref, bh_ref,
              h_ref, s_ref, soff_ref, feat_ref):
    x = x_ref[...]
    h = _elu(jax.lax.dot_general(x, w1_ref[...], (((1,), (1,)), ((), ())),
                                 preferred_element_type=jnp.float32) + b1_ref[...])
    h = _elu(jax.lax.dot_general(h, w2_ref[...], (((1,), (1,)), ((), ())),
                                 preferred_element_type=jnp.float32) + b2_ref[...])
    h = _bn_in(h, g1_ref[...], be1_ref[...])
    h_ref[...] = h
    s = jax.lax.dot_general(h, ws_ref[...], (((1,), (1,)), ((), ())),
                            preferred_element_type=jnp.float32) + bs_ref[...]
    s_ref[...] = s
    col = jax.lax.broadcasted_iota(jnp.int32, (1, 8), 1)
    off = jnp.where(col < SPACE, batchf_ref[...] * 1000.0, 0.0)
    soff_ref[...] = s + off
    feat_ref[...] = jax.lax.dot_general(h, wh_ref[...], (((1,), (1,)), ((), ())),
                                        preferred_element_type=jnp.float32) + bh_ref[...]


def _post_body(h_ref, s_ref, mean_ref, max_ref, dist_ref,
               wo1_ref, wo2a_ref, wo2b_ref, bo2_ref,
               wpa_ref, wps_ref, wpc_ref, bp1_ref, wp2_ref, bp2_ref,
               g2_ref, be2_ref, p_ref, loss_ref):
    h = h_ref[...]
    dn = (((1,), (1,)), ((), ()))
    xgn = (jax.lax.dot_general(h, wo1_ref[...], dn, preferred_element_type=jnp.float32)
           + jax.lax.dot_general(mean_ref[...], wo2a_ref[...], dn, preferred_element_type=jnp.float32)
           + jax.lax.dot_general(max_ref[...], wo2b_ref[...], dn, preferred_element_type=jnp.float32)
           + bo2_ref[...])
    p = _elu(jax.lax.dot_general(xgn, wpa_ref[...], dn, preferred_element_type=jnp.float32)
             + jax.lax.dot_general(s_ref[...], wps_ref[...], dn, preferred_element_type=jnp.float32)
             + jax.lax.dot_general(h, wpc_ref[...], dn, preferred_element_type=jnp.float32)
             + bp1_ref[...])
    p = _elu(jax.lax.dot_general(p, wp2_ref[...], dn, preferred_element_type=jnp.float32)
             + bp2_ref[...])
    p_ref[...] = _bn_in(p, g2_ref[...], be2_ref[...])
    loss_ref[...] = jnp.mean(jnp.sqrt(dist_ref[...] + 1e-12), keepdims=True)


def _knn(s, k, chunk):
    n = s.shape[0]
    s2 = jnp.sum(s * s, axis=1)

    def body(q):
        d = jnp.sum(q * q, axis=1)[:, None] - 2.0 * (q @ s.T) + s2[None, :]
        nd, idx = jax.lax.top_k(-d, k)
        return -nd, idx

    qs = s.reshape(n // chunk, chunk, s.shape[1])
    d, idx = jax.lax.map(body, qs)
    return jnp.maximum(d.reshape(n, k), 0.0), idx.reshape(n, k)


def kernel(x, batch, original_coords, W_pre1, b_pre1, W_pre2, b_pre2, gamma1,
           beta1, W_s, b_s, W_h, b_h, W_out1, W_out2, b_out2, W_post1, b_post1,
           W_post2, b_post2, gamma2, beta2):
    f32 = jnp.float32
    batchf = batch.astype(f32).reshape(N, 1)
    ws_pad = jnp.zeros((8, D), f32).at[:SPACE].set(W_s)
    bs_pad = jnp.zeros((1, 8), f32).at[0, :SPACE].set(b_s)

    h, s, s_off, feat = pl.pallas_call(
        _pre_body,
        out_shape=(
            jax.ShapeDtypeStruct((N, D), f32),
            jax.ShapeDtypeStruct((N, 8), f32),
            jax.ShapeDtypeStruct((N, 8), f32),
            jax.ShapeDtypeStruct((N, D), f32),
        ),
    )(x, batchf, W_pre1, b_pre1.reshape(1, D), W_pre2, b_pre2.reshape(1, D),
      gamma1.reshape(1, D), beta1.reshape(1, D), ws_pad, bs_pad, W_h,
      b_h.reshape(1, D))

    dist_sq, idx = _knn(s_off, K, 500)
    w = jnp.exp(-10.0 * dist_sq)
    nb = feat[idx]
    msg = nb * w[..., None]
    mean_agg = jnp.mean(msg, axis=1)
    max_agg = jnp.max(msg, axis=1)

    p, loss = pl.pallas_call(
        _post_body,
        out_shape=(
            jax.ShapeDtypeStruct((N, D), f32),
            jax.ShapeDtypeStruct((1, 1), f32),
        ),
    )(h, s, mean_agg, max_agg, dist_sq,
      W_out1, W_out2[:, :D], W_out2[:, D:], b_out2.reshape(1, D),
      W_post1[:, :D], jnp.zeros((D, 8), f32).at[:, :SPACE].set(W_post1[:, D:D + SPACE]),
      W_post1[:, D + SPACE:], b_post1.reshape(1, D), W_post2,
      b_post2.reshape(1, D), gamma2.reshape(1, D), beta2.reshape(1, D))

    return (p, loss.reshape(()), jnp.float32(0.0))


# speedup vs baseline: 1.0195x; 1.0195x over previous
import jax
import jax.numpy as jnp
from jax.experimental import pallas as pl
from jax.experimental.pallas import tpu as pltpu

N = 10000
IN_CH = 128
D = 32
SPACE = 3
K = 40
NB = 4
QB = 200


def _bn(h, g, b):
    m = jnp.mean(h, axis=0)
    v = jnp.var(h, axis=0)
    return (h - m) / jnp.sqrt(v + 1e-5) * g + b


def _dist_body(q_ref, qq_ref, s_ref, s2_ref, d_ref):
    m = jax.lax.dot_general(q_ref[...], s_ref[...], (((1,), (1,)), ((), ())),
                            preferred_element_type=jnp.float32)
    d_ref[...] = (qq_ref[...] - 2.0 * m) + s2_ref[...]


def _dist_matrix(s_off):
    s2 = jnp.sum(s_off * s_off, axis=1)
    qq = s2.reshape(N, 1)
    s2r = s2.reshape(1, N)
    return pl.pallas_call(
        _dist_body,
        grid=(N // QB,),
        in_specs=[
            pl.BlockSpec((QB, SPACE), lambda i: (i, 0)),
            pl.BlockSpec((QB, 1), lambda i: (i, 0)),
            pl.BlockSpec((N, SPACE), lambda i: (0, 0)),
            pl.BlockSpec((1, N), lambda i: (0, 0)),
        ],
        out_specs=pl.BlockSpec((QB, N), lambda i: (i, 0)),
        out_shape=jax.ShapeDtypeStruct((N, N), jnp.float32),
    )(s_off, qq, s_off, s2r)


def kernel(x, batch, original_coords, W_pre1, b_pre1, W_pre2, b_pre2, gamma1,
           beta1, W_s, b_s, W_h, b_h, W_out1, W_out2, b_out2, W_post1, b_post1,
           W_post2, b_post2, gamma2, beta2):
    h = jax.nn.elu(x @ W_pre1.T + b_pre1)
    h = jax.nn.elu(h @ W_pre2.T + b_pre2)
    h = _bn(h, gamma1, beta1)
    x_input = h
    s = h @ W_s.T + b_s
    feat = h @ W_h.T + b_h
    s_off = s + batch[:, None].astype(s.dtype) * 1000.0

    d = _dist_matrix(s_off)
    negd, idx = jax.lax.top_k(-d, K)
    dist_sq = jnp.maximum(-negd, 0.0)

    w = jnp.exp(-10.0 * dist_sq)
    nb = feat[idx]
    msg = nb * w[..., None]
    mean_agg = jnp.mean(msg, axis=1)
    max_agg = jnp.max(msg, axis=1)
    xgn = h @ W_out1.T + jnp.concatenate([mean_agg, max_agg], axis=1) @ W_out2.T + b_out2
    loss_reg = jnp.mean(jnp.sqrt(dist_sq + 1e-12))
    cat = jnp.concatenate([xgn, s, x_input], axis=1)
    p = jax.nn.elu(cat @ W_post1.T + b_post1)
    p = jax.nn.elu(p @ W_post2.T + b_post2)
    p = _bn(p, gamma2, beta2)
    return (p, loss_reg, jnp.float32(0.0))


# speedup vs baseline: 3.0997x; 3.0404x over previous
import functools

import jax
import jax.numpy as jnp
from jax import lax
from jax.experimental import pallas as pl
from jax.experimental.pallas import tpu as pltpu
from jax.experimental.pallas import tpu_sc as plsc

N = 10000
IN_CH = 128
D = 32
SPACE = 3
K = 40
NB = 4
QB = 200
NC, NS = 2, 16
NW = NC * NS
QPW = 320
NPAD = NW * QPW
CAP = 4096
NV = N // 16


def _dist_body(q_ref, qq_ref, s_ref, s2_ref, d_ref):
    m = jax.lax.dot_general(q_ref[...], s_ref[...], (((1,), (1,)), ((), ())),
                            preferred_element_type=jnp.float32)
    d_ref[...] = (qq_ref[...] - 2.0 * m) + s2_ref[...]


def _dist_matrix(s_off):
    s2 = jnp.sum(s_off * s_off, axis=1)
    qq = s2.reshape(N, 1)
    s2r = s2.reshape(1, N)
    return pl.pallas_call(
        _dist_body,
        grid=(N // QB,),
        in_specs=[
            pl.BlockSpec((QB, SPACE), lambda i: (i, 0)),
            pl.BlockSpec((QB, 1), lambda i: (i, 0)),
            pl.BlockSpec((N, SPACE), lambda i: (0, 0)),
            pl.BlockSpec((1, N), lambda i: (0, 0)),
        ],
        out_specs=pl.BlockSpec((QB, N), lambda i: (i, 0)),
        out_shape=jax.ShapeDtypeStruct((N, N), jnp.float32),
    )(s_off, qq, s_off, s2r)


def _wexp(x, i32, f32):
    x = jnp.maximum(x, -87.0)
    t = x * 1.4426950408889634
    n = (t + jnp.where(t >= 0, 0.5, -0.5)).astype(i32)
    nf = n.astype(f32)
    r = (x - nf * 0.693359375) + nf * 2.12194440e-4
    p = 1.0 / 720.0
    p = p * r + 1.0 / 120.0
    p = p * r + 1.0 / 24.0
    p = p * r + 1.0 / 6.0
    p = p * r + 0.5
    p = p * r + 1.0
    p = p * r + 1.0
    scale = plsc.bitcast((n + 127) << 23, f32)
    return p * scale


def _sc_body(d_hbm, feat_hbm, qs_hbm, qe_hbm,
             mean_hbm, max_hbm, dist_hbm, idx_hbm,
             dbuf, skey, sd, sidx, seld, selidx, gidx, wbuf, rows, tmpd, tmpi, gsem,
             sstart, send, omean, omax, odist, oidx):
    i32 = jnp.int32
    u32 = jnp.uint32
    f32 = jnp.float32
    wid = lax.axis_index("s") * NC + lax.axis_index("c")
    base = wid * QPW
    qcnt = jnp.minimum(jnp.int32(QPW), jnp.int32(N) - base)

    pltpu.sync_copy(qs_hbm.at[pl.ds(base, QPW)], sstart.at[pl.ds(0, QPW)])
    pltpu.sync_copy(qe_hbm.at[pl.ds(base, QPW)], send.at[pl.ds(0, QPW)])

    def popcnt(m):
        return jnp.max(plsc.all_reduce_population_count(m))

    def per_query(i, t_carry):
        q = base + i
        pltpu.sync_copy(d_hbm.at[q], dbuf)
        i0 = lax.div(i, jnp.int32(16)) * 16
        lane = i - i0
        lm = jnp.arange(16, dtype=i32) == lane
        st = jnp.max(jnp.where(lm, sstart[pl.ds(i0, 16)], jnp.int32(-1)))
        en = jnp.max(jnp.where(lm, send[pl.ds(i0, 16)], jnp.int32(-1)))
        vs = lax.div(st, jnp.int32(16))
        ve = lax.div(en + jnp.int32(15), jnp.int32(16))

        def compact_pass(T):
            def cb(j, carry):
                ptr, craw = carry
                v = dbuf[pl.ds(j * 16, 16)]
                g = jnp.arange(16, dtype=i32) + j * 16
                m_raw = (v < T) & (g >= st) & (g < en)
                m = m_raw & (ptr < CAP)
                cpc = popcnt(m)

                @pl.when(cpc > 0)
                def _():
                    u = plsc.bitcast(v, i32)
                    key = u ^ jnp.where(v < 0.0, jnp.int32(0x7FFFFFFF),
                                        jnp.int32(0))
                    plsc.store_compressed(skey.at[pl.ds(ptr, 16)], key, mask=m)
                    plsc.store_compressed(sd.at[pl.ds(ptr, 16)], v, mask=m)
                    plsc.store_compressed(sidx.at[pl.ds(ptr, 16)], g, mask=m)

                return ptr + cpc, craw + popcnt(m_raw)
            return lax.fori_loop(vs, ve, cb, (jnp.int32(0), jnp.int32(0)))

        ptr0, craw0 = compact_pass(t_carry)

        def acond(stt):
            _T, _lo, _hi, _p, c, it = stt
            return ((c < K) | (c > CAP)) & (it < 48)

        def abody(stt):
            T, lo, hi, _p, c, it = stt
            lo2 = jnp.where(c < K, T, lo)
            hi2 = jnp.where(c > CAP, T, hi)
            have_hi = hi2 < 3.9e9
            have_lo = lo2 > -0.9e9
            mid = 0.5 * (lo2 + hi2)
            T_up = jnp.where(have_hi, mid,
                             jnp.where(T > 0, T * 4.0 + 1.0, T * 0.25 + 1.0))
            T_dn = jnp.where(have_lo, mid,
                             jnp.where(T > 0, T * 0.25 - 1.0, T * 4.0 - 1.0))
            T2 = jnp.where(c < K, T_up, T_dn)
            p2, c2 = compact_pass(T2)
            return (T2, lo2, hi2, p2, c2, it + 1)

        T, _, _, ptr, _, _ = lax.while_loop(
            acond, abody,
            (t_carry, jnp.float32(-1e9), jnp.float32(4e9), ptr0, craw0,
             jnp.int32(0)))

        skey[pl.ds(ptr, 16)] = jnp.full((16,), 0x7FFFFFFF, i32)
        nvec = lax.div(ptr + jnp.int32(15), jnp.int32(16))

        sbias = jnp.int32(-2147483648)

        def bit_body(bb, Ru):
            bit = jnp.int32(1) << (jnp.int32(31) - bb)
            test_s = (Ru | bit) ^ sbias

            def ccount(j, acc):
                kv = skey[pl.ds(j * 16, 16)]
                return acc + plsc.all_reduce_population_count(kv < test_s)
            cc = jnp.max(lax.fori_loop(jnp.int32(0), nvec, ccount,
                                       jnp.zeros((16,), i32)))
            return jnp.where(cc <= K - 1, Ru | bit, Ru)

        Ru = lax.fori_loop(0, 32, bit_body, jnp.int32(0))
        R = Ru ^ sbias

        def dcount(j, acc):
            kv = skey[pl.ds(j * 16, 16)]
            return acc + plsc.all_reduce_population_count(kv < R)
        c_lt = jnp.max(lax.fori_loop(jnp.int32(0), nvec, dcount,
                                     jnp.zeros((16,), i32)))
        m_eq = K - c_lt

        def dbody(j, carry):
            ptr2, m_rem = carry
            kv = skey[pl.ds(j * 16, 16)]
            dv = sd[pl.ds(j * 16, 16)]
            iv = sidx[pl.ds(j * 16, 16)]
            m_lt = kv < R
            plsc.store_compressed(seld.at[pl.ds(ptr2, 16)], dv, mask=m_lt)
            plsc.store_compressed(selidx.at[pl.ds(ptr2, 16)], iv, mask=m_lt)
            ptr2 = ptr2 + popcnt(m_lt)
            meq = kv == R
            neq = popcnt(meq)
            take = jnp.minimum(m_rem, neq)

            @pl.when(take > 0)
            def _():
                plsc.store_compressed(tmpd.at[pl.ds(0, 16)], dv, mask=meq)
                plsc.store_compressed(tmpi.at[pl.ds(0, 16)], iv, mask=meq)
                mt = jnp.arange(16, dtype=i32) < take
                plsc.store_compressed(seld.at[pl.ds(ptr2, 16)],
                                      tmpd[pl.ds(0, 16)], mask=mt)
                plsc.store_compressed(selidx.at[pl.ds(ptr2, 16)],
                                      tmpi[pl.ds(0, 16)], mask=mt)

            return ptr2 + take, m_rem - take

        lax.fori_loop(jnp.int32(0), nvec, dbody, (jnp.int32(0), m_eq))

        gidx[pl.ds(0, 16)] = selidx[pl.ds(0, 16)]
        gidx[pl.ds(16, 16)] = selidx[pl.ds(16, 16)]
        gidx[pl.ds(24, 16)] = selidx[pl.ds(24, 16)]
        pltpu.async_copy(feat_hbm.at[gidx], rows, gsem).wait()

        d0 = jnp.maximum(seld[pl.ds(0, 16)], 0.0)
        d1 = jnp.maximum(seld[pl.ds(16, 16)], 0.0)
        d2 = jnp.maximum(seld[pl.ds(24, 16)], 0.0)
        wbuf[pl.ds(0, 16)] = _wexp(-10.0 * d0, i32, f32)
        wbuf[pl.ds(16, 16)] = _wexp(-10.0 * d1, i32, f32)
        wbuf[pl.ds(24, 16)] = _wexp(-10.0 * d2, i32, f32)
        odist[pl.ds(i * K + 0, 16)] = d0
        odist[pl.ds(i * K + 16, 16)] = d1
        odist[pl.ds(i * K + 24, 16)] = d2
        oidx[pl.ds(i * K + 0, 16)] = selidx[pl.ds(0, 16)]
        oidx[pl.ds(i * K + 16, 16)] = selidx[pl.ds(16, 16)]
        oidx[pl.ds(i * K + 24, 16)] = selidx[pl.ds(24, 16)]

        acc0 = jnp.zeros((16,), f32)
        acc1 = jnp.zeros((16,), f32)
        mx0 = None
        mx1 = None
        for k in range(K):
            bw = plsc.load_gather(wbuf, [jnp.full((16,), k, i32)])
            r0 = rows[k, pl.ds(0, 16)]
            r1 = rows[k, pl.ds(16, 16)]
            m0 = r0 * bw
            m1 = r1 * bw
            acc0 = acc0 + m0
            acc1 = acc1 + m1
            if k == 0:
                mx0, mx1 = m0, m1
            else:
                mx0 = jnp.maximum(mx0, m0)
                mx1 = jnp.maximum(mx1, m1)
        omean[pl.ds(i * D + 0, 16)] = acc0 / f32(K)
        omean[pl.ds(i * D + 16, 16)] = acc1 / f32(K)
        omax[pl.ds(i * D + 0, 16)] = mx0
        omax[pl.ds(i * D + 16, 16)] = mx1

        maxd = jnp.maximum(jnp.max(seld[pl.ds(0, 16)]),
                           jnp.maximum(jnp.max(seld[pl.ds(16, 16)]),
                                       jnp.max(seld[pl.ds(24, 16)])))
        return maxd * 4.0 + 1e-2

    lax.fori_loop(jnp.int32(0), qcnt, per_query, jnp.float32(1.0))

    pltpu.sync_copy(omean, mean_hbm.at[pl.ds(base * D, QPW * D)])
    pltpu.sync_copy(omax, max_hbm.at[pl.ds(base * D, QPW * D)])
    pltpu.sync_copy(odist, dist_hbm.at[pl.ds(base * K, QPW * K)])
    pltpu.sync_copy(oidx, idx_hbm.at[pl.ds(base * K, QPW * K)])


def _sc_select_aggregate(d, feat, qs, qe):
    f32 = jnp.float32
    i32 = jnp.int32
    mesh = plsc.VectorSubcoreMesh(core_axis_name="c", subcore_axis_name="s")
    fn = pl.kernel(
        _sc_body,
        out_type=(
            jax.ShapeDtypeStruct((NPAD * D,), f32),
            jax.ShapeDtypeStruct((NPAD * D,), f32),
            jax.ShapeDtypeStruct((NPAD * K,), f32),
            jax.ShapeDtypeStruct((NPAD * K,), i32),
        ),
        mesh=mesh,
        compiler_params=pltpu.CompilerParams(needs_layout_passes=False),
        scratch_types=[
            pltpu.VMEM((N,), f32),
            pltpu.VMEM((CAP + 32,), i32),
            pltpu.VMEM((CAP + 32,), f32),
            pltpu.VMEM((CAP + 32,), i32),
            pltpu.VMEM((64,), f32),
            pltpu.VMEM((64,), i32),
            pltpu.VMEM((K,), i32),
            pltpu.VMEM((64,), f32),
            pltpu.VMEM((K, 128), f32),
            pltpu.VMEM((32,), f32),
            pltpu.VMEM((32,), i32),
            pltpu.SemaphoreType.DMA,
            pltpu.VMEM((QPW + 16,), i32),
            pltpu.VMEM((QPW + 16,), i32),
            pltpu.VMEM((QPW * D,), f32),
            pltpu.VMEM((QPW * D,), f32),
            pltpu.VMEM((QPW * K,), f32),
            pltpu.VMEM((QPW * K,), i32),
        ],
    )
    mean_f, max_f, dist_f, idx_f = fn(d, feat, qs, qe)
    return (mean_f.reshape(NPAD, D), max_f.reshape(NPAD, D),
            dist_f.reshape(NPAD, K), idx_f.reshape(NPAD, K))


def _elu_p(v):
    return jnp.where(v > 0, v, jnp.exp(jnp.minimum(v, 0.0)) - 1.0)


def _bn_in(h, g, b):
    m = jnp.mean(h, axis=0, keepdims=True)
    v = jnp.mean((h - m) ** 2, axis=0, keepdims=True)
    return (h - m) / jnp.sqrt(v + 1e-5) * g + b


def _post_body(h_ref, s_ref, mean_ref, max_ref, dist_ref,
               wo1_ref, wo2a_ref, wo2b_ref, bo2_ref,
               wpa_ref, wps_ref, wpc_ref, bp1_ref, wp2_ref, bp2_ref,
               g2_ref, be2_ref, p_ref, loss_ref):
    h = h_ref[...]
    dn = (((1,), (1,)), ((), ()))
    f32 = jnp.float32
    xgn = (jax.lax.dot_general(h, wo1_ref[...], dn, preferred_element_type=f32)
           + jax.lax.dot_general(mean_ref[...], wo2a_ref[...], dn, preferred_element_type=f32)
           + jax.lax.dot_general(max_ref[...], wo2b_ref[...], dn, preferred_element_type=f32)
           + bo2_ref[...])
    p = _elu_p(jax.lax.dot_general(xgn, wpa_ref[...], dn, preferred_element_type=f32)
               + jax.lax.dot_general(s_ref[...], wps_ref[...], dn, preferred_element_type=f32)
               + jax.lax.dot_general(h, wpc_ref[...], dn, preferred_element_type=f32)
               + bp1_ref[...])
    p = _elu_p(jax.lax.dot_general(p, wp2_ref[...], dn, preferred_element_type=f32)
               + bp2_ref[...])
    p_ref[...] = _bn_in(p, g2_ref[...], be2_ref[...])
    loss_ref[...] = jnp.mean(jnp.sqrt(dist_ref[...] + 1e-12), keepdims=True)


def _bn(h, g, b):
    m = jnp.mean(h, axis=0)
    v = jnp.var(h, axis=0)
    return (h - m) / jnp.sqrt(v + 1e-5) * g + b


def kernel(x, batch, original_coords, W_pre1, b_pre1, W_pre2, b_pre2, gamma1,
           beta1, W_s, b_s, W_h, b_h, W_out1, W_out2, b_out2, W_post1, b_post1,
           W_post2, b_post2, gamma2, beta2):
    f32 = jnp.float32
    i32 = jnp.int32

    h = jax.nn.elu(x @ W_pre1.T + b_pre1)
    h = jax.nn.elu(h @ W_pre2.T + b_pre2)
    h = _bn(h, gamma1, beta1)
    s = h @ W_s.T + b_s
    feat = h @ W_h.T + b_h
    s_off = s + batch[:, None].astype(s.dtype) * 1000.0

    ar = jnp.arange(NB, dtype=batch.dtype)
    bstart = jnp.searchsorted(batch, ar, side="left").astype(i32)
    bend = jnp.searchsorted(batch, ar, side="right").astype(i32)
    qs = bstart[batch]
    qe = bend[batch]
    ln = qe - qs
    qs = jnp.where(ln >= K, qs, 0)
    qe = jnp.where(ln >= K, qe, N)
    qs = jnp.concatenate([qs, jnp.zeros((NPAD - N,), i32)])
    qe = jnp.concatenate([qe, jnp.full((NPAD - N,), N, i32)])

    d = _dist_matrix(s_off)
    feat128 = jnp.zeros((N, 128), f32).at[:, :D].set(feat)
    _scm, _scx, dist_f, idx_f = _sc_select_aggregate(d, feat128, qs, qe)
    dist_sq = dist_f[:N]
    idx = idx_f[:N]
    w = jnp.exp(-10.0 * dist_sq)
    nb = feat[idx]
    msg = nb * w[..., None]
    mean_agg = jnp.mean(msg, axis=1)
    max_agg = jnp.max(msg, axis=1)

    p, loss = pl.pallas_call(
        _post_body,
        out_shape=(
            jax.ShapeDtypeStruct((N, D), f32),
            jax.ShapeDtypeStruct((1, 1), f32),
        ),
    )(h, jnp.zeros((N, 8), f32).at[:, :SPACE].set(s), mean_agg, max_agg,
      dist_sq,
      W_out1, W_out2[:, :D], W_out2[:, D:], b_out2.reshape(1, D),
      W_post1[:, :D],
      jnp.zeros((D, 8), f32).at[:, :SPACE].set(W_post1[:, D:D + SPACE]),
      W_post1[:, D + SPACE:], b_post1.reshape(1, D), W_post2,
      b_post2.reshape(1, D), gamma2.reshape(1, D), beta2.reshape(1, D))

    return (p, loss.reshape(()), jnp.float32(0.0))


# speedup vs baseline: 3.2182x; 1.0382x over previous
import functools

import jax
import jax.numpy as jnp
from jax import lax
from jax.experimental import pallas as pl
from jax.experimental.pallas import tpu as pltpu
from jax.experimental.pallas import tpu_sc as plsc

N = 10000
IN_CH = 128
D = 32
SPACE = 3
K = 40
NB = 4
QB = 200
NC, NS = 2, 16
NW = NC * NS
QPW = 320
NPAD = NW * QPW
CAP = 4096
NV = N // 16


def _dist_body(q_ref, qq_ref, s_ref, s2_ref, d_ref):
    m = jax.lax.dot_general(q_ref[...], s_ref[...], (((1,), (1,)), ((), ())),
                            preferred_element_type=jnp.float32)
    d_ref[...] = (qq_ref[...] - 2.0 * m) + s2_ref[...]


def _dist_matrix(s_off):
    s2 = jnp.sum(s_off * s_off, axis=1)
    qq = s2.reshape(N, 1)
    s2r = s2.reshape(1, N)
    return pl.pallas_call(
        _dist_body,
        grid=(N // QB,),
        in_specs=[
            pl.BlockSpec((QB, SPACE), lambda i: (i, 0)),
            pl.BlockSpec((QB, 1), lambda i: (i, 0)),
            pl.BlockSpec((N, SPACE), lambda i: (0, 0)),
            pl.BlockSpec((1, N), lambda i: (0, 0)),
        ],
        out_specs=pl.BlockSpec((QB, N), lambda i: (i, 0)),
        out_shape=jax.ShapeDtypeStruct((N, N), jnp.float32),
    )(s_off, qq, s_off, s2r)


def _wexp(x, i32, f32):
    x = jnp.maximum(x, -87.0)
    t = x * 1.4426950408889634
    n = (t + jnp.where(t >= 0, 0.5, -0.5)).astype(i32)
    nf = n.astype(f32)
    r = (x - nf * 0.693359375) + nf * 2.12194440e-4
    p = 1.0 / 720.0
    p = p * r + 1.0 / 120.0
    p = p * r + 1.0 / 24.0
    p = p * r + 1.0 / 6.0
    p = p * r + 0.5
    p = p * r + 1.0
    p = p * r + 1.0
    scale = plsc.bitcast((n + 127) << 23, f32)
    return p * scale


def _sc_body(d_hbm, qs_hbm, qe_hbm,
             dist_hbm, idx_hbm,
             dbuf, skey, sd, sidx, seld, selidx, tmpd, tmpi,
             sstart, send, odist, oidx):
    i32 = jnp.int32
    u32 = jnp.uint32
    f32 = jnp.float32
    wid = lax.axis_index("s") * NC + lax.axis_index("c")
    base = wid * QPW
    qcnt = jnp.minimum(jnp.int32(QPW), jnp.int32(N) - base)

    pltpu.sync_copy(qs_hbm.at[pl.ds(base, QPW)], sstart.at[pl.ds(0, QPW)])
    pltpu.sync_copy(qe_hbm.at[pl.ds(base, QPW)], send.at[pl.ds(0, QPW)])

    def popcnt(m):
        return jnp.max(plsc.all_reduce_population_count(m))

    def per_query(i, t_carry):
        q = base + i
        pltpu.sync_copy(d_hbm.at[q], dbuf)
        i0 = lax.div(i, jnp.int32(16)) * 16
        lane = i - i0
        lm = jnp.arange(16, dtype=i32) == lane
        st = jnp.max(jnp.where(lm, sstart[pl.ds(i0, 16)], jnp.int32(-1)))
        en = jnp.max(jnp.where(lm, send[pl.ds(i0, 16)], jnp.int32(-1)))
        vs = lax.div(st, jnp.int32(16))
        ve = lax.div(en + jnp.int32(15), jnp.int32(16))

        def compact_pass(T):
            def cb(j, carry):
                ptr, craw = carry
                v = dbuf[pl.ds(j * 16, 16)]
                g = jnp.arange(16, dtype=i32) + j * 16
                m_raw = (v < T) & (g >= st) & (g < en)
                m = m_raw & (ptr < CAP)
                cpc = popcnt(m)

                @pl.when(cpc > 0)
                def _():
                    u = plsc.bitcast(v, i32)
                    key = u ^ jnp.where(v < 0.0, jnp.int32(0x7FFFFFFF),
                                        jnp.int32(0))
                    plsc.store_compressed(skey.at[pl.ds(ptr, 16)], key, mask=m)
                    plsc.store_compressed(sd.at[pl.ds(ptr, 16)], v, mask=m)
                    plsc.store_compressed(sidx.at[pl.ds(ptr, 16)], g, mask=m)

                return ptr + cpc, craw + popcnt(m_raw)
            return lax.fori_loop(vs, ve, cb, (jnp.int32(0), jnp.int32(0)))

        ptr0, craw0 = compact_pass(t_carry)

        def acond(stt):
            _T, _lo, _hi, _p, c, it = stt
            return ((c < K) | (c > CAP)) & (it < 48)

        def abody(stt):
            T, lo, hi, _p, c, it = stt
            lo2 = jnp.where(c < K, T, lo)
            hi2 = jnp.where(c > CAP, T, hi)
            have_hi = hi2 < 3.9e9
            have_lo = lo2 > -0.9e9
            mid = 0.5 * (lo2 + hi2)
            T_up = jnp.where(have_hi, mid,
                             jnp.where(T > 0, T * 4.0 + 1.0, T * 0.25 + 1.0))
            T_dn = jnp.where(have_lo, mid,
                             jnp.where(T > 0, T * 0.25 - 1.0, T * 4.0 - 1.0))
            T2 = jnp.where(c < K, T_up, T_dn)
            p2, c2 = compact_pass(T2)
            return (T2, lo2, hi2, p2, c2, it + 1)

        T, _, _, ptr, _, _ = lax.while_loop(
            acond, abody,
            (t_carry, jnp.float32(-1e9), jnp.float32(4e9), ptr0, craw0,
             jnp.int32(0)))

        skey[pl.ds(ptr, 16)] = jnp.full((16,), 0x7FFFFFFF, i32)
        nvec = lax.div(ptr + jnp.int32(15), jnp.int32(16))

        sbias = jnp.int32(-2147483648)

        def bit_body(bb, Ru):
            bit = jnp.int32(1) << (jnp.int32(31) - bb)
            test_s = (Ru | bit) ^ sbias

            def ccount(j, acc):
                kv = skey[pl.ds(j * 16, 16)]
                return acc + plsc.all_reduce_population_count(kv < test_s)
            cc = jnp.max(lax.fori_loop(jnp.int32(0), nvec, ccount,
                                       jnp.zeros((16,), i32)))
            return jnp.where(cc <= K - 1, Ru | bit, Ru)

        Ru = lax.fori_loop(0, 32, bit_body, jnp.int32(0))
        R = Ru ^ sbias

        def dcount(j, acc):
            kv = skey[pl.ds(j * 16, 16)]
            return acc + plsc.all_reduce_population_count(kv < R)
        c_lt = jnp.max(lax.fori_loop(jnp.int32(0), nvec, dcount,
                                     jnp.zeros((16,), i32)))
        m_eq = K - c_lt

        def dbody(j, carry):
            ptr2, m_rem = carry
            kv = skey[pl.ds(j * 16, 16)]
            dv = sd[pl.ds(j * 16, 16)]
            iv = sidx[pl.ds(j * 16, 16)]
            m_lt = kv < R
            plsc.store_compressed(seld.at[pl.ds(ptr2, 16)], dv, mask=m_lt)
            plsc.store_compressed(selidx.at[pl.ds(ptr2, 16)], iv, mask=m_lt)
            ptr2 = ptr2 + popcnt(m_lt)
            meq = kv == R
            neq = popcnt(meq)
            take = jnp.minimum(m_rem, neq)

            @pl.when(take > 0)
            def _():
                plsc.store_compressed(tmpd.at[pl.ds(0, 16)], dv, mask=meq)
                plsc.store_compressed(tmpi.at[pl.ds(0, 16)], iv, mask=meq)
                mt = jnp.arange(16, dtype=i32) < take
                plsc.store_compressed(seld.at[pl.ds(ptr2, 16)],
                                      tmpd[pl.ds(0, 16)], mask=mt)
                plsc.store_compressed(selidx.at[pl.ds(ptr2, 16)],
                                      tmpi[pl.ds(0, 16)], mask=mt)

            return ptr2 + take, m_rem - take

        lax.fori_loop(jnp.int32(0), nvec, dbody, (jnp.int32(0), m_eq))

        d0 = jnp.maximum(seld[pl.ds(0, 16)], 0.0)
        d1 = jnp.maximum(seld[pl.ds(16, 16)], 0.0)
        d2 = jnp.maximum(seld[pl.ds(24, 16)], 0.0)
        odist[pl.ds(i * K + 0, 16)] = d0
        odist[pl.ds(i * K + 16, 16)] = d1
        odist[pl.ds(i * K + 24, 16)] = d2
        oidx[pl.ds(i * K + 0, 16)] = selidx[pl.ds(0, 16)]
        oidx[pl.ds(i * K + 16, 16)] = selidx[pl.ds(16, 16)]
        oidx[pl.ds(i * K + 24, 16)] = selidx[pl.ds(24, 16)]

        maxd = jnp.maximum(jnp.max(seld[pl.ds(0, 16)]),
                           jnp.maximum(jnp.max(seld[pl.ds(16, 16)]),
                                       jnp.max(seld[pl.ds(24, 16)])))
        return maxd * 4.0 + 1e-2

    lax.fori_loop(jnp.int32(0), qcnt, per_query, jnp.float32(1.0))

    pltpu.sync_copy(odist, dist_hbm.at[pl.ds(base * K, QPW * K)])
    pltpu.sync_copy(oidx, idx_hbm.at[pl.ds(base * K, QPW * K)])


def _sc_select(d, qs, qe):
    f32 = jnp.float32
    i32 = jnp.int32
    mesh = plsc.VectorSubcoreMesh(core_axis_name="c", subcore_axis_name="s")
    fn = pl.kernel(
        _sc_body,
        out_type=(
            jax.ShapeDtypeStruct((NPAD * K,), f32),
            jax.ShapeDtypeStruct((NPAD * K,), i32),
        ),
        mesh=mesh,
        compiler_params=pltpu.CompilerParams(needs_layout_passes=False),
        scratch_types=[
            pltpu.VMEM((N,), f32),
            pltpu.VMEM((CAP + 32,), i32),
            pltpu.VMEM((CAP + 32,), f32),
            pltpu.VMEM((CAP + 32,), i32),
            pltpu.VMEM((64,), f32),
            pltpu.VMEM((64,), i32),
            pltpu.VMEM((32,), f32),
            pltpu.VMEM((32,), i32),
            pltpu.VMEM((QPW + 16,), i32),
            pltpu.VMEM((QPW + 16,), i32),
            pltpu.VMEM((QPW * K,), f32),
            pltpu.VMEM((QPW * K,), i32),
        ],
    )
    dist_f, idx_f = fn(d, qs, qe)
    return dist_f.reshape(NPAD, K), idx_f.reshape(NPAD, K)


def _elu_p(v):
    return jnp.where(v > 0, v, jnp.exp(jnp.minimum(v, 0.0)) - 1.0)


def _bn_in(h, g, b):
    m = jnp.mean(h, axis=0, keepdims=True)
    v = jnp.mean((h - m) ** 2, axis=0, keepdims=True)
    return (h - m) / jnp.sqrt(v + 1e-5) * g + b


def _post_body(h_ref, s_ref, mean_ref, max_ref, dist_ref,
               wo1_ref, wo2a_ref, wo2b_ref, bo2_ref,
               wpa_ref, wps_ref, wpc_ref, bp1_ref, wp2_ref, bp2_ref,
               g2_ref, be2_ref, p_ref, loss_ref):
    h = h_ref[...]
    dn = (((1,), (1,)), ((), ()))
    f32 = jnp.float32
    xgn = (jax.lax.dot_general(h, wo1_ref[...], dn, preferred_element_type=f32)
           + jax.lax.dot_general(mean_ref[...], wo2a_ref[...], dn, preferred_element_type=f32)
           + jax.lax.dot_general(max_ref[...], wo2b_ref[...], dn, preferred_element_type=f32)
           + bo2_ref[...])
    p = _elu_p(jax.lax.dot_general(xgn, wpa_ref[...], dn, preferred_element_type=f32)
               + jax.lax.dot_general(s_ref[...], wps_ref[...], dn, preferred_element_type=f32)
               + jax.lax.dot_general(h, wpc_ref[...], dn, preferred_element_type=f32)
               + bp1_ref[...])
    p = _elu_p(jax.lax.dot_general(p, wp2_ref[...], dn, preferred_element_type=f32)
               + bp2_ref[...])
    p_ref[...] = _bn_in(p, g2_ref[...], be2_ref[...])
    loss_ref[...] = jnp.mean(jnp.sqrt(dist_ref[...] + 1e-12), keepdims=True)


def _bn(h, g, b):
    m = jnp.mean(h, axis=0)
    v = jnp.var(h, axis=0)
    return (h - m) / jnp.sqrt(v + 1e-5) * g + b


def kernel(x, batch, original_coords, W_pre1, b_pre1, W_pre2, b_pre2, gamma1,
           beta1, W_s, b_s, W_h, b_h, W_out1, W_out2, b_out2, W_post1, b_post1,
           W_post2, b_post2, gamma2, beta2):
    f32 = jnp.float32
    i32 = jnp.int32

    h = jax.nn.elu(x @ W_pre1.T + b_pre1)
    h = jax.nn.elu(h @ W_pre2.T + b_pre2)
    h = _bn(h, gamma1, beta1)
    s = h @ W_s.T + b_s
    feat = h @ W_h.T + b_h
    s_off = s + batch[:, None].astype(s.dtype) * 1000.0

    ar = jnp.arange(NB, dtype=batch.dtype)
    bstart = jnp.searchsorted(batch, ar, side="left").astype(i32)
    bend = jnp.searchsorted(batch, ar, side="right").astype(i32)
    qs = bstart[batch]
    qe = bend[batch]
    ln = qe - qs
    qs = jnp.where(ln >= K, qs, 0)
    qe = jnp.where(ln >= K, qe, N)
    qs = jnp.concatenate([qs, jnp.zeros((NPAD - N,), i32)])
    qe = jnp.concatenate([qe, jnp.full((NPAD - N,), N, i32)])

    d = _dist_matrix(s_off)
    dist_f, idx_f = _sc_select(d, qs, qe)
    dist_sq = dist_f[:N]
    idx = idx_f[:N]
    w = jnp.exp(-10.0 * dist_sq)
    nb = feat[idx]
    msg = nb * w[..., None]
    mean_agg = jnp.mean(msg, axis=1)
    max_agg = jnp.max(msg, axis=1)

    p, loss = pl.pallas_call(
        _post_body,
        out_shape=(
            jax.ShapeDtypeStruct((N, D), f32),
            jax.ShapeDtypeStruct((1, 1), f32),
        ),
    )(h, jnp.zeros((N, 8), f32).at[:, :SPACE].set(s), mean_agg, max_agg,
      dist_sq,
      W_out1, W_out2[:, :D], W_out2[:, D:], b_out2.reshape(1, D),
      W_post1[:, :D],
      jnp.zeros((D, 8), f32).at[:, :SPACE].set(W_post1[:, D:D + SPACE]),
      W_post1[:, D + SPACE:], b_post1.reshape(1, D), W_post2,
      b_post2.reshape(1, D), gamma2.reshape(1, D), beta2.reshape(1, D))

    return (p, loss.reshape(()), jnp.float32(0.0))


# speedup vs baseline: 3.2189x; 1.0002x over previous
import jax
import jax.numpy as jnp
from jax import lax
from jax.experimental import pallas as pl
from jax.experimental.pallas import tpu as pltpu
from jax.experimental.pallas import tpu_sc as plsc

N = 10000
IN_CH = 128
D = 32
SPACE = 3
K = 40
NB = 4
QB = 200
NC, NS = 2, 16
NW = NC * NS
QPW = 320
NPAD = NW * QPW
CAP = 4096


def _dist_body(q_ref, qq_ref, s_ref, s2_ref, d_ref):
    m = jax.lax.dot_general(q_ref[...], s_ref[...], (((1,), (1,)), ((), ())),
                            preferred_element_type=jnp.float32)
    d_ref[...] = (qq_ref[...] - 2.0 * m) + s2_ref[...]


def _dist_matrix(s_off):
    s2 = jnp.sum(s_off * s_off, axis=1)
    qq = s2.reshape(N, 1)
    s2r = s2.reshape(1, N)
    return pl.pallas_call(
        _dist_body,
        grid=(N // QB,),
        in_specs=[
            pl.BlockSpec((QB, SPACE), lambda i: (i, 0)),
            pl.BlockSpec((QB, 1), lambda i: (i, 0)),
            pl.BlockSpec((N, SPACE), lambda i: (0, 0)),
            pl.BlockSpec((1, N), lambda i: (0, 0)),
        ],
        out_specs=pl.BlockSpec((QB, N), lambda i: (i, 0)),
        out_shape=jax.ShapeDtypeStruct((N, N), jnp.float32),
    )(s_off, qq, s_off, s2r)


def _sc_body(d_hbm, qs_hbm, qe_hbm,
             dist_hbm, idx_hbm,
             dbuf, skey, sd, sidx, seld, selidx, tmpd, tmpi,
             sstart, send, odist, oidx):
    i32 = jnp.int32
    f32 = jnp.float32
    wid = lax.axis_index("s") * NC + lax.axis_index("c")
    base = wid * QPW
    qcnt = jnp.minimum(jnp.int32(QPW), jnp.int32(N) - base)

    pltpu.sync_copy(qs_hbm.at[pl.ds(base, QPW)], sstart.at[pl.ds(0, QPW)])
    pltpu.sync_copy(qe_hbm.at[pl.ds(base, QPW)], send.at[pl.ds(0, QPW)])

    def popcnt(m):
        return jnp.max(plsc.all_reduce_population_count(m))

    def per_query(i, t_carry):
        q = base + i
        pltpu.sync_copy(d_hbm.at[q], dbuf)
        i0 = lax.div(i, jnp.int32(16)) * 16
        lane = i - i0
        lm = jnp.arange(16, dtype=i32) == lane
        st = jnp.max(jnp.where(lm, sstart[pl.ds(i0, 16)], jnp.int32(-1)))
        en = jnp.max(jnp.where(lm, send[pl.ds(i0, 16)], jnp.int32(-1)))
        vs = lax.div(st, jnp.int32(16))
        ve = lax.div(en + jnp.int32(15), jnp.int32(16))

        def compact_pass(T):
            def cb(j, carry):
                ptr, craw = carry
                v = dbuf[pl.ds(j * 16, 16)]
                g = jnp.arange(16, dtype=i32) + j * 16
                m_raw = (v < T) & (g >= st) & (g < en)
                m = m_raw & (ptr < CAP)
                cpc = popcnt(m)

                @pl.when(cpc > 0)
                def _():
                    u = plsc.bitcast(v, i32)
                    key = u ^ jnp.where(v < 0.0, jnp.int32(0x7FFFFFFF),
                                        jnp.int32(0))
                    plsc.store_compressed(skey.at[pl.ds(ptr, 16)], key, mask=m)
                    plsc.store_compressed(sd.at[pl.ds(ptr, 16)], v, mask=m)
                    plsc.store_compressed(sidx.at[pl.ds(ptr, 16)], g, mask=m)

                return ptr + cpc, craw + popcnt(m_raw)
            return lax.fori_loop(vs, ve, cb, (jnp.int32(0), jnp.int32(0)))

        ptr0, craw0 = compact_pass(t_carry)

        def acond(stt):
            _T, _lo, _hi, _p, c, it = stt
            return ((c < K) | (c > CAP)) & (it < 48)

        def abody(stt):
            T, lo, hi, _p, c, it = stt
            lo2 = jnp.where(c < K, T, lo)
            hi2 = jnp.where(c > CAP, T, hi)
            have_hi = hi2 < 3.9e9
            have_lo = lo2 > -0.9e9
            mid = 0.5 * (lo2 + hi2)
            T_up = jnp.where(have_hi, mid,
                             jnp.where(T > 0, T * 4.0 + 1.0, T * 0.25 + 1.0))
            T_dn = jnp.where(have_lo, mid,
                             jnp.where(T > 0, T * 0.25 - 1.0, T * 4.0 - 1.0))
            T2 = jnp.where(c < K, T_up, T_dn)
            p2, c2 = compact_pass(T2)
            return (T2, lo2, hi2, p2, c2, it + 1)

        T, _, _, ptr, _, _ = lax.while_loop(
            acond, abody,
            (t_carry, jnp.float32(-1e9), jnp.float32(4e9), ptr0, craw0,
             jnp.int32(0)))

        skey[pl.ds(ptr, 16)] = jnp.full((16,), 0x7FFFFFFF, i32)
        nvec = lax.div(ptr + jnp.int32(15), jnp.int32(16))

        sbias = jnp.int32(-2147483648)

        def bit_body(bb, Ru):
            bit = jnp.int32(1) << (jnp.int32(31) - bb)
            test_s = (Ru | bit) ^ sbias

            def ccount(j, acc):
                kv = skey[pl.ds(j * 16, 16)]
                return acc + plsc.all_reduce_population_count(kv < test_s)
            cc = jnp.max(lax.fori_loop(jnp.int32(0), nvec, ccount,
                                       jnp.zeros((16,), i32)))
            return jnp.where(cc <= K - 1, Ru | bit, Ru)

        Ru = lax.fori_loop(0, 32, bit_body, jnp.int32(0))
        R = Ru ^ sbias

        def dcount(j, acc):
            kv = skey[pl.ds(j * 16, 16)]
            return acc + plsc.all_reduce_population_count(kv < R)
        c_lt = jnp.max(lax.fori_loop(jnp.int32(0), nvec, dcount,
                                     jnp.zeros((16,), i32)))
        m_eq = K - c_lt

        def dbody(j, carry):
            ptr2, m_rem = carry
            kv = skey[pl.ds(j * 16, 16)]
            dv = sd[pl.ds(j * 16, 16)]
            iv = sidx[pl.ds(j * 16, 16)]
            m_lt = kv < R
            plsc.store_compressed(seld.at[pl.ds(ptr2, 16)], dv, mask=m_lt)
            plsc.store_compressed(selidx.at[pl.ds(ptr2, 16)], iv, mask=m_lt)
            ptr2 = ptr2 + popcnt(m_lt)
            meq = kv == R
            neq = popcnt(meq)
            take = jnp.minimum(m_rem, neq)

            @pl.when(take > 0)
            def _():
                plsc.store_compressed(tmpd.at[pl.ds(0, 16)], dv, mask=meq)
                plsc.store_compressed(tmpi.at[pl.ds(0, 16)], iv, mask=meq)
                mt = jnp.arange(16, dtype=i32) < take
                plsc.store_compressed(seld.at[pl.ds(ptr2, 16)],
                                      tmpd[pl.ds(0, 16)], mask=mt)
                plsc.store_compressed(selidx.at[pl.ds(ptr2, 16)],
                                      tmpi[pl.ds(0, 16)], mask=mt)

            return ptr2 + take, m_rem - take

        lax.fori_loop(jnp.int32(0), nvec, dbody, (jnp.int32(0), m_eq))

        d0 = jnp.maximum(seld[pl.ds(0, 16)], 0.0)
        d1 = jnp.maximum(seld[pl.ds(16, 16)], 0.0)
        d2 = jnp.maximum(seld[pl.ds(24, 16)], 0.0)
        odist[pl.ds(i * K + 0, 16)] = d0
        odist[pl.ds(i * K + 16, 16)] = d1
        odist[pl.ds(i * K + 24, 16)] = d2
        oidx[pl.ds(i * K + 0, 16)] = selidx[pl.ds(0, 16)]
        oidx[pl.ds(i * K + 16, 16)] = selidx[pl.ds(16, 16)]
        oidx[pl.ds(i * K + 24, 16)] = selidx[pl.ds(24, 16)]

        maxd = jnp.maximum(jnp.max(seld[pl.ds(0, 16)]),
                           jnp.maximum(jnp.max(seld[pl.ds(16, 16)]),
                                       jnp.max(seld[pl.ds(24, 16)])))
        return maxd * 4.0 + 1e-2

    lax.fori_loop(jnp.int32(0), qcnt, per_query, jnp.float32(1.0))

    pltpu.sync_copy(odist, dist_hbm.at[pl.ds(base * K, QPW * K)])
    pltpu.sync_copy(oidx, idx_hbm.at[pl.ds(base * K, QPW * K)])


def _sc_select(d, qs, qe):
    f32 = jnp.float32
    i32 = jnp.int32
    mesh = plsc.VectorSubcoreMesh(core_axis_name="c", subcore_axis_name="s")
    fn = pl.kernel(
        _sc_body,
        out_type=(
            jax.ShapeDtypeStruct((NPAD * K,), f32),
            jax.ShapeDtypeStruct((NPAD * K,), i32),
        ),
        mesh=mesh,
        compiler_params=pltpu.CompilerParams(needs_layout_passes=False),
        scratch_types=[
            pltpu.VMEM((N,), f32),
            pltpu.VMEM((CAP + 32,), i32),
            pltpu.VMEM((CAP + 32,), f32),
            pltpu.VMEM((CAP + 32,), i32),
            pltpu.VMEM((64,), f32),
            pltpu.VMEM((64,), i32),
            pltpu.VMEM((32,), f32),
            pltpu.VMEM((32,), i32),
            pltpu.VMEM((QPW + 16,), i32),
            pltpu.VMEM((QPW + 16,), i32),
            pltpu.VMEM((QPW * K,), f32),
            pltpu.VMEM((QPW * K,), i32),
        ],
    )
    dist_f, idx_f = fn(d, qs, qe)
    return dist_f.reshape(NPAD, K), idx_f.reshape(NPAD, K)


def _elu_p(v):
    return jnp.where(v > 0, v, jnp.exp(jnp.minimum(v, 0.0)) - 1.0)


def _bn_in(h, g, b):
    m = jnp.mean(h, axis=0, keepdims=True)
    v = jnp.mean((h - m) ** 2, axis=0, keepdims=True)
    return (h - m) / jnp.sqrt(v + 1e-5) * g + b


def _post_body(h_ref, s_ref, mean_ref, max_ref, dist_ref,
               wo1_ref, wo2a_ref, wo2b_ref, bo2_ref,
               wpa_ref, wps_ref, wpc_ref, bp1_ref, wp2_ref, bp2_ref,
               g2_ref, be2_ref, p_ref, loss_ref):
    h = h_ref[...]
    dn = (((1,), (1,)), ((), ()))
    f32 = jnp.float32
    xgn = (jax.lax.dot_general(h, wo1_ref[...], dn, preferred_element_type=f32)
           + jax.lax.dot_general(mean_ref[...], wo2a_ref[...], dn, preferred_element_type=f32)
           + jax.lax.dot_general(max_ref[...], wo2b_ref[...], dn, preferred_element_type=f32)
           + bo2_ref[...])
    p = _elu_p(jax.lax.dot_general(xgn, wpa_ref[...], dn, preferred_element_type=f32)
               + jax.lax.dot_general(s_ref[...], wps_ref[...], dn, preferred_element_type=f32)
               + jax.lax.dot_general(h, wpc_ref[...], dn, preferred_element_type=f32)
               + bp1_ref[...])
    p = _elu_p(jax.lax.dot_general(p, wp2_ref[...], dn, preferred_element_type=f32)
               + bp2_ref[...])
    p_ref[...] = _bn_in(p, g2_ref[...], be2_ref[...])
    loss_ref[...] = jnp.mean(jnp.sqrt(dist_ref[...] + 1e-12), keepdims=True)


def _bn(h, g, b):
    m = jnp.mean(h, axis=0)
    v = jnp.var(h, axis=0)
    return (h - m) / jnp.sqrt(v + 1e-5) * g + b


def kernel(x, batch, original_coords, W_pre1, b_pre1, W_pre2, b_pre2, gamma1,
           beta1, W_s, b_s, W_h, b_h, W_out1, W_out2, b_out2, W_post1, b_post1,
           W_post2, b_post2, gamma2, beta2):
    f32 = jnp.float32
    i32 = jnp.int32

    h = jax.nn.elu(x @ W_pre1.T + b_pre1)
    h = jax.nn.elu(h @ W_pre2.T + b_pre2)
    h = _bn(h, gamma1, beta1)
    s = h @ W_s.T + b_s
    feat = h @ W_h.T + b_h
    s_off = s + batch[:, None].astype(s.dtype) * 1000.0

    ar = jnp.arange(NB, dtype=batch.dtype)
    bstart = jnp.searchsorted(batch, ar, side="left").astype(i32)
    bend = jnp.searchsorted(batch, ar, side="right").astype(i32)
    qs = bstart[batch]
    qe = bend[batch]
    ln = qe - qs
    qs = jnp.where(ln >= K, qs, 0)
    qe = jnp.where(ln >= K, qe, N)
    qs = jnp.concatenate([qs, jnp.zeros((NPAD - N,), i32)])
    qe = jnp.concatenate([qe, jnp.full((NPAD - N,), N, i32)])

    d = _dist_matrix(s_off)
    dist_f, idx_f = _sc_select(d, qs, qe)
    dist_sq = dist_f[:N]
    idx = idx_f[:N]
    w = jnp.exp(-10.0 * dist_sq)
    nb = feat[idx]
    msg = nb * w[..., None]
    mean_agg = jnp.mean(msg, axis=1)
    max_agg = jnp.max(msg, axis=1)

    p, loss = pl.pallas_call(
        _post_body,
        out_shape=(
            jax.ShapeDtypeStruct((N, D), f32),
            jax.ShapeDtypeStruct((1, 1), f32),
        ),
    )(h, jnp.zeros((N, 8), f32).at[:, :SPACE].set(s), mean_agg, max_agg,
      dist_sq,
      W_out1, W_out2[:, :D], W_out2[:, D:], b_out2.reshape(1, D),
      W_post1[:, :D],
      jnp.zeros((D, 8), f32).at[:, :SPACE].set(W_post1[:, D:D + SPACE]),
      W_post1[:, D + SPACE:], b_post1.reshape(1, D), W_post2,
      b_post2.reshape(1, D), gamma2.reshape(1, D), beta2.reshape(1, D))

    return (p, loss.reshape(()), jnp.float32(0.0))
